# baseline (speedup 1.0000x reference)
import functools

import jax
import jax.numpy as jnp
from jax import lax
from jax.experimental import pallas as pl
from jax.experimental.pallas import tpu as pltpu
from jax.experimental.pallas import tpu_sc as plsc

D = 8192
D_OUT = 4096
N_FIXED = 8
BLK = 256

NC = 2
NS = 16
NW = NC * NS
ROWS_PER_W = D_OUT // NW
WR_ROW = 128
WR_STRIDE = D // WR_ROW


def _sc_fixed_kernel(wr_hbm, attrs_hbm, b_hbm, out_hbm,
                     idx_v, rows_v, attrs_v, b_v, out_v, sem):
    c = lax.axis_index("c")
    s = lax.axis_index("s")
    wid = s * NC + c
    base = wid * ROWS_PER_W
    iota = lax.iota(jnp.int32, 16)
    for ch in range(ROWS_PER_W // 16):
        idx_v[pl.ds(ch * 16, 16)] = (base + ch * 16 + iota) * WR_STRIDE
    pltpu.sync_copy(attrs_hbm.at[pl.ds(0, 16)], attrs_v)
    pltpu.sync_copy(b_hbm.at[pl.ds(base, ROWS_PER_W)], b_v)
    pltpu.async_copy(wr_hbm.at[idx_v], rows_v, sem).wait()
    am = jnp.where(iota < N_FIXED, attrs_v[...], 0.0)
    for ch in range(ROWS_PER_W // 16):
        acc = b_v[pl.ds(ch * 16, 16)]
        for r in range(16):
            v = rows_v[ch * 16 + r, pl.ds(0, 16)]
            s = jnp.sum(v * am)
            acc = acc + jnp.where(iota == r, s, 0.0)
        out_v[pl.ds(ch * 16, 16)] = acc
    pltpu.sync_copy(out_v, out_hbm.at[pl.ds(base, ROWS_PER_W)])


def _tc_matvec_kernel(xp_ref, w_ref, out_ref):
    xs = pltpu.roll(xp_ref[...], N_FIXED, axis=1)
    out_ref[...] = jax.lax.dot_general(
        xs, w_ref[...], (((1,), (1,)), ((), ())),
        preferred_element_type=jnp.float32)


@functools.partial(jax.jit, static_argnames=())
def kernel(x, attrs_init, W, b):
    xp = jnp.pad(x, ((0, 0), (0, N_FIXED)))

    sc_fn = pl.kernel(
        _sc_fixed_kernel,
        out_type=jax.ShapeDtypeStruct((D_OUT,), jnp.float32),
        mesh=plsc.VectorSubcoreMesh(
            core_axis_name="c", subcore_axis_name="s",
            num_cores=NC, num_subcores=NS),
        scratch_types=[
            pltpu.VMEM((ROWS_PER_W,), jnp.int32),
            pltpu.VMEM((ROWS_PER_W, WR_ROW), jnp.float32),
            pltpu.VMEM((16,), jnp.float32),
            pltpu.VMEM((ROWS_PER_W,), jnp.float32),
            pltpu.VMEM((ROWS_PER_W,), jnp.float32),
            pltpu.SemaphoreType.DMA,
        ],
        compiler_params=pltpu.CompilerParams(needs_layout_passes=False),
    )
    sc_out = sc_fn(W.reshape(-1, WR_ROW), attrs_init.reshape(-1), b)

    tc_out = pl.pallas_call(
        _tc_matvec_kernel,
        grid=(D_OUT // BLK,),
        in_specs=[
            pl.BlockSpec((1, D), lambda i: (0, 0)),
            pl.BlockSpec((BLK, D), lambda i: (i, 0)),
        ],
        out_specs=pl.BlockSpec((1, BLK), lambda i: (0, i)),
        out_shape=jax.ShapeDtypeStruct((1, D_OUT), jnp.float32),
    )(xp, W)

    return tc_out + sc_out.reshape(1, D_OUT)


# speedup vs baseline: 4.6062x; 4.6062x over previous
import functools

import jax
import jax.numpy as jnp
from jax.experimental import pallas as pl
from jax.experimental.pallas import tpu as pltpu

D = 8192
D_OUT = 4096
N_FIXED = 8
BLK = 256
NBLOCKS = D_OUT // BLK
NBUF = 4


def _ffm_kernel(xp_ref, attrs_ref, b_ref, w_hbm, out_ref, *scratch):
    bufs = scratch[:NBUF]
    sems = scratch[NBUF:]

    def w_copy(block, slot):
        return pltpu.make_async_copy(
            w_hbm.at[pl.ds(block * BLK, BLK), :], bufs[slot], sems[slot])

    for t in range(NBUF):
        w_copy(t, t).start()

    xs = pltpu.roll(xp_ref[...], N_FIXED, axis=1)
    col = jax.lax.broadcasted_iota(jnp.int32, (1, D), 1)
    inp = jnp.where(col < N_FIXED, attrs_ref[...], xs)

    for i in range(NBLOCKS):
        t = i % NBUF
        w_copy(i, t).wait()
        acc = jax.lax.dot_general(
            inp, bufs[t][...], (((1,), (1,)), ((), ())),
            preferred_element_type=jnp.float32)
        out_ref[:, pl.ds(i * BLK, BLK)] = acc + b_ref[:, pl.ds(i * BLK, BLK)]
        if i + NBUF < NBLOCKS:
            w_copy(i + NBUF, t).start()


@functools.partial(jax.jit, static_argnames=())
def kernel(x, attrs_init, W, b):
    xp = jnp.pad(x, ((0, 0), (0, N_FIXED)))
    b2 = b.reshape(1, D_OUT)
    out = pl.pallas_call(
        _ffm_kernel,
        in_specs=[
            pl.BlockSpec((1, D), lambda: (0, 0)),
            pl.BlockSpec((1, D), lambda: (0, 0)),
            pl.BlockSpec((1, D_OUT), lambda: (0, 0)),
            pl.BlockSpec(memory_space=pltpu.HBM),
        ],
        out_specs=pl.BlockSpec((1, D_OUT), lambda: (0, 0)),
        out_shape=jax.ShapeDtypeStruct((1, D_OUT), jnp.float32),
        scratch_shapes=(
            [pltpu.VMEM((BLK, D), jnp.float32) for _ in range(NBUF)]
            + [pltpu.SemaphoreType.DMA for _ in range(NBUF)]
        ),
    )(xp, attrs_init, b2, W)
    return out


# speedup vs baseline: 4.6968x; 1.0197x over previous
import functools

import jax
import jax.numpy as jnp
from jax.experimental import pallas as pl
from jax.experimental.pallas import tpu as pltpu

D = 8192
D_OUT = 4096
N_FIXED = 8
BLK = 128
NBLOCKS = D_OUT // BLK
NBUF = 8


def _ffm_kernel(xp_ref, attrs_ref, b_ref, w_hbm, out_ref, *scratch):
    bufs = scratch[:NBUF]
    sems = scratch[NBUF:]

    def w_copy(block, slot):
        return pltpu.make_async_copy(
            w_hbm.at[pl.ds(block * BLK, BLK), :], bufs[slot], sems[slot])

    for t in range(NBUF):
        w_copy(t, t).start()

    xs = pltpu.roll(xp_ref[...], N_FIXED, axis=1)
    col = jax.lax.broadcasted_iota(jnp.int32, (1, D), 1)
    inp = jnp.where(col < N_FIXED, attrs_ref[...], xs)

    for i in range(NBLOCKS):
        t = i % NBUF
        w_copy(i, t).wait()
        acc = jax.lax.dot_general(
            inp, bufs[t][...], (((1,), (1,)), ((), ())),
            preferred_element_type=jnp.float32)
        out_ref[:, pl.ds(i * BLK, BLK)] = acc + b_ref[:, pl.ds(i * BLK, BLK)]
        if i + NBUF < NBLOCKS:
            w_copy(i + NBUF, t).start()


@functools.partial(jax.jit, static_argnames=())
def kernel(x, attrs_init, W, b):
    xp = jnp.pad(x, ((0, 0), (0, N_FIXED)))
    b2 = b.reshape(1, D_OUT)
    out = pl.pallas_call(
        _ffm_kernel,
        in_specs=[
            pl.BlockSpec((1, D), lambda: (0, 0)),
            pl.BlockSpec((1, D), lambda: (0, 0)),
            pl.BlockSpec((1, D_OUT), lambda: (0, 0)),
            pl.BlockSpec(memory_space=pltpu.HBM),
        ],
        out_specs=pl.BlockSpec((1, D_OUT), lambda: (0, 0)),
        out_shape=jax.ShapeDtypeStruct((1, D_OUT), jnp.float32),
        scratch_shapes=(
            [pltpu.VMEM((BLK, D), jnp.float32) for _ in range(NBUF)]
            + [pltpu.SemaphoreType.DMA for _ in range(NBUF)]
        ),
    )(xp, attrs_init, b2, W)
    return out


# speedup vs baseline: 4.7220x; 1.0054x over previous
import functools

import jax
import jax.numpy as jnp
from jax.experimental import pallas as pl
from jax.experimental.pallas import tpu as pltpu

D = 8192
D_OUT = 4096
N_FIXED = 8
BLK = 256


def _ffm_kernel(xp_ref, attrs_ref, w_ref, b_ref, out_ref):
    xs = pltpu.roll(xp_ref[...], N_FIXED, axis=1)
    col = jax.lax.broadcasted_iota(jnp.int32, (1, D), 1)
    inp = jnp.where(col < N_FIXED, attrs_ref[...], xs)
    acc = jax.lax.dot_general(
        inp, w_ref[...], (((1,), (1,)), ((), ())),
        preferred_element_type=jnp.float32)
    out_ref[...] = acc + b_ref[...]


@functools.partial(jax.jit, static_argnames=())
def kernel(x, attrs_init, W, b):
    xp = jnp.pad(x, ((0, 0), (0, N_FIXED)))
    b2 = b.reshape(1, D_OUT)
    grid = (D_OUT // BLK,)
    out = pl.pallas_call(
        _ffm_kernel,
        grid=grid,
        in_specs=[
            pl.BlockSpec((1, D), lambda i: (0, 0)),
            pl.BlockSpec((1, D), lambda i: (0, 0)),
            pl.BlockSpec((BLK, D), lambda i: (i, 0)),
            pl.BlockSpec((1, BLK), lambda i: (0, i)),
        ],
        out_specs=pl.BlockSpec((1, BLK), lambda i: (0, i)),
        out_shape=jax.ShapeDtypeStruct((1, D_OUT), jnp.float32),
    )(xp, attrs_init, W, b2)
    return out


# speedup vs baseline: 4.9476x; 1.0478x over previous
import functools

import jax
import jax.numpy as jnp
from jax.experimental import pallas as pl
from jax.experimental.pallas import tpu as pltpu

D = 8192
D_OUT = 4096
N_FIXED = 8
BLK = 256


def _ffm_kernel(xp_ref, attrs_ref, w_ref, b_ref, out_ref):
    xs = pltpu.roll(xp_ref[...], N_FIXED, axis=1)
    col = jax.lax.broadcasted_iota(jnp.int32, (1, D), 1)
    inp = jnp.where(col < N_FIXED, attrs_ref[...], xs)
    acc = jax.lax.dot_general(
        inp, w_ref[...], (((1,), (1,)), ((), ())),
        preferred_element_type=jnp.float32)
    i = pl.program_id(0)
    out_ref[...] = acc + b_ref[:, pl.ds(i * BLK, BLK)]


@functools.partial(jax.jit, static_argnames=())
def kernel(x, attrs_init, W, b):
    xp = jnp.pad(x, ((0, 0), (0, N_FIXED)))
    b2 = b.reshape(1, D_OUT)
    grid = (D_OUT // BLK,)
    out = pl.pallas_call(
        _ffm_kernel,
        grid=grid,
        in_specs=[
            pl.BlockSpec((1, D), lambda i: (0, 0)),
            pl.BlockSpec((1, D), lambda i: (0, 0)),
            pl.BlockSpec((BLK, D), lambda i: (i, 0)),
            pl.BlockSpec((1, D_OUT), lambda i: (0, 0)),
        ],
        out_specs=pl.BlockSpec((1, BLK), lambda i: (0, i)),
        out_shape=jax.ShapeDtypeStruct((1, D_OUT), jnp.float32),
    )(xp, attrs_init, W, b2)
    return out


# speedup vs baseline: 4.9512x; 1.0007x over previous
import functools

import jax
import jax.numpy as jnp
from jax.experimental import pallas as pl
from jax.experimental.pallas import tpu as pltpu

D = 8192
D_OUT = 4096
N_FIXED = 8
BLK = 256


def _ffm_kernel(xp_ref, attrs_ref, w_ref, b_ref, out_ref):
    xs = pltpu.roll(xp_ref[...], N_FIXED, axis=1)
    col = jax.lax.broadcasted_iota(jnp.int32, (1, D), 1)
    inp = jnp.where(col < N_FIXED, attrs_ref[...], xs)
    acc = jax.lax.dot_general(
        inp, w_ref[...], (((1,), (1,)), ((), ())),
        preferred_element_type=jnp.float32)
    i = pl.program_id(0)
    out_ref[:, pl.ds(i * BLK, BLK)] = acc + b_ref[:, pl.ds(i * BLK, BLK)]


@functools.partial(jax.jit, static_argnames=())
def kernel(x, attrs_init, W, b):
    xp = jnp.pad(x, ((0, 0), (0, N_FIXED)))
    b2 = b.reshape(1, D_OUT)
    grid = (D_OUT // BLK,)
    out = pl.pallas_call(
        _ffm_kernel,
        grid=grid,
        in_specs=[
            pl.BlockSpec((1, D), lambda i: (0, 0)),
            pl.BlockSpec((1, D), lambda i: (0, 0)),
            pl.BlockSpec((BLK, D), lambda i: (i, 0)),
            pl.BlockSpec((1, D_OUT), lambda i: (0, 0)),
        ],
        out_specs=pl.BlockSpec((1, D_OUT), lambda i: (0, 0)),
        out_shape=jax.ShapeDtypeStruct((1, D_OUT), jnp.float32),
    )(xp, attrs_init, W, b2)
    return out
